# Initial kernel scaffold; baseline (speedup 1.0000x reference)
import jax
import jax.numpy as jnp
from jax.experimental import pallas as pl


def kernel(input_ids, embed_table, W, b):
    raise NotImplementedError("write your pallas kernel here")


# speedup vs baseline: 110.0471x; 110.0471x over previous
import functools

import jax
import jax.numpy as jnp
from jax import lax
from jax.experimental import pallas as pl
from jax.experimental.pallas import tpu as pltpu
from jax.experimental.pallas import tpu_sc as plsc

B = 16384
L = 200
V = 200
D = 16
C = 2

_info = plsc.get_sparse_core_info()
NC, NS, LANES = _info.num_cores, _info.num_subcores, _info.num_lanes
NW = NC * NS
ROWS_PER_W = B // NW
CHUNK_ROWS = 64
NCHUNK = ROWS_PER_W // CHUNK_ROWS
NBLK = CHUNK_ROWS // LANES
VPAD = 208


def _sc_body(ids_hbm, tab_hbm, w_hbm, b_hbm, out_hbm,
             ids_a, ids_b, tab_v, w_v, b_v, m0_v, m1_v, out_v,
             sem_a, sem_b, sem_s):
    wid = lax.axis_index("s") * NC + lax.axis_index("c")
    base = wid * ROWS_PER_W

    pltpu.async_copy(tab_hbm, tab_v, sem_s).wait()
    pltpu.async_copy(w_hbm, w_v, sem_s).wait()
    pltpu.async_copy(b_hbm, b_v, sem_s).wait()

    lanes = lax.iota(jnp.int32, LANES)
    zero16 = jnp.zeros((LANES,), jnp.int32)
    one16 = zero16 + 1

    for vc in range(VPAD // LANES):
        vflat = jnp.minimum(lanes + vc * LANES, V - 1) * D

        def mbody(dd, carry, vflat=vflat):
            m0, m1 = carry
            dvec = jnp.full((LANES,), dd, dtype=jnp.int32)
            col = plsc.load_gather(tab_v, [vflat + dvec])
            w0 = plsc.load_gather(w_v, [dvec])
            w1 = plsc.load_gather(w_v, [dvec + D])
            return (m0 + col * w0, m1 + col * w1)

        m0, m1 = lax.fori_loop(
            0, D, mbody,
            (jnp.zeros((LANES,), jnp.float32), jnp.zeros((LANES,), jnp.float32)))
        m0_v[pl.ds(vc * LANES, LANES)] = m0
        m1_v[pl.ds(vc * LANES, LANES)] = m1

    b0 = plsc.load_gather(b_v, [zero16])
    b1 = plsc.load_gather(b_v, [one16])
    inv_l = jnp.full((LANES,), 1.0 / L, dtype=jnp.float32)

    bufs = (ids_a, ids_b)
    sems = (sem_a, sem_b)
    handles = [None, None]
    handles[0] = pltpu.async_copy(
        ids_hbm.at[pl.ds(base * L, CHUNK_ROWS * L)], ids_a, sem_a)
    for ch in range(NCHUNK):
        cur = ch % 2
        handles[cur].wait()
        if ch + 1 < NCHUNK:
            nxt = (ch + 1) % 2
            handles[nxt] = pltpu.async_copy(
                ids_hbm.at[pl.ds((base + (ch + 1) * CHUNK_ROWS) * L,
                                 CHUNK_ROWS * L)],
                bufs[nxt], sems[nxt])
        ids_buf = bufs[cur]
        for blk in range(NBLK):
            rowbase = (lanes + blk * LANES) * L

            def lbody(ll, carry, ids_buf=ids_buf, rowbase=rowbase):
                a0, a1 = carry
                lvec = jnp.full((LANES,), ll, dtype=jnp.int32)
                idv = plsc.load_gather(ids_buf, [rowbase + lvec])
                g0 = plsc.load_gather(m0_v, [idv])
                g1 = plsc.load_gather(m1_v, [idv])
                return (a0 + g0, a1 + g1)

            a0, a1 = lax.fori_loop(
                0, L, lbody,
                (jnp.zeros((LANES,), jnp.float32),
                 jnp.zeros((LANES,), jnp.float32)))
            a0 = a0 * inv_l + b0
            a1 = a1 * inv_l + b1
            rloc2 = (lanes + blk * LANES + ch * CHUNK_ROWS) * C
            plsc.store_scatter(out_v, [rloc2], a0)
            plsc.store_scatter(out_v, [rloc2 + 1], a1)

    pltpu.async_copy(out_v, out_hbm.at[pl.ds(base * C, ROWS_PER_W * C)],
                     sem_s).wait()


_sc_kernel = functools.partial(
    pl.kernel,
    out_type=jax.ShapeDtypeStruct((B * C,), jnp.float32),
    mesh=plsc.VectorSubcoreMesh(core_axis_name="c", subcore_axis_name="s"),
    compiler_params=pltpu.CompilerParams(needs_layout_passes=False),
    scratch_types=[
        pltpu.VMEM((CHUNK_ROWS * L,), jnp.int32),
        pltpu.VMEM((CHUNK_ROWS * L,), jnp.int32),
        pltpu.VMEM((V * D,), jnp.float32),
        pltpu.VMEM((C * D,), jnp.float32),
        pltpu.VMEM((LANES,), jnp.float32),
        pltpu.VMEM((VPAD,), jnp.float32),
        pltpu.VMEM((VPAD,), jnp.float32),
        pltpu.VMEM((ROWS_PER_W * C,), jnp.float32),
        pltpu.SemaphoreType.DMA,
        pltpu.SemaphoreType.DMA,
        pltpu.SemaphoreType.DMA,
    ],
)(_sc_body)


def kernel(input_ids, embed_table, W, b):
    ids = input_ids.astype(jnp.int32).reshape(B * L)
    tab = embed_table.astype(jnp.float32).reshape(V * D)
    w = W.astype(jnp.float32).reshape(C * D)
    b_pad = jnp.zeros((LANES,), jnp.float32).at[:C].set(b.astype(jnp.float32))
    out = _sc_kernel(ids, tab, w, b_pad)
    return out.reshape(B, C)


# speedup vs baseline: 125.7320x; 1.1425x over previous
import functools

import jax
import jax.numpy as jnp
from jax import lax
from jax.experimental import pallas as pl
from jax.experimental.pallas import tpu as pltpu
from jax.experimental.pallas import tpu_sc as plsc

B = 16384
L = 200
V = 200
D = 16
C = 2

_info = plsc.get_sparse_core_info()
NC, NS, LANES = _info.num_cores, _info.num_subcores, _info.num_lanes
NW = NC * NS
ROWS_PER_W = B // NW
CHUNK_ROWS = 64
NCHUNK = ROWS_PER_W // CHUNK_ROWS
NBLK = CHUNK_ROWS // LANES
VPAD = 208


def _sc_body(ids_hbm, tab_hbm, w_hbm, b_hbm, out_hbm,
             ids_a, ids_b, tab_v, w_v, b_v, m0_v, m1_v, out_v,
             sem_a, sem_b, sem_s):
    wid = lax.axis_index("s") * NC + lax.axis_index("c")
    base = wid * ROWS_PER_W

    pltpu.async_copy(tab_hbm, tab_v, sem_s).wait()
    pltpu.async_copy(w_hbm, w_v, sem_s).wait()
    pltpu.async_copy(b_hbm, b_v, sem_s).wait()

    lanes = lax.iota(jnp.int32, LANES)
    zero16 = jnp.zeros((LANES,), jnp.int32)
    one16 = zero16 + 1

    for vc in range(VPAD // LANES):
        vflat = jnp.minimum(lanes + vc * LANES, V - 1) * D

        def mbody(dd, carry, vflat=vflat):
            m0, m1 = carry
            dvec = jnp.full((LANES,), dd, dtype=jnp.int32)
            col = plsc.load_gather(tab_v, [vflat + dvec])
            w0 = plsc.load_gather(w_v, [dvec])
            w1 = plsc.load_gather(w_v, [dvec + D])
            return (m0 + col * w0, m1 + col * w1)

        m0, m1 = lax.fori_loop(
            0, D, mbody,
            (jnp.zeros((LANES,), jnp.float32), jnp.zeros((LANES,), jnp.float32)))
        m0_v[pl.ds(vc * LANES, LANES)] = m0
        m1_v[pl.ds(vc * LANES, LANES)] = m1

    b0 = plsc.load_gather(b_v, [zero16])
    b1 = plsc.load_gather(b_v, [one16])
    inv_l = jnp.full((LANES,), 1.0 / L, dtype=jnp.float32)

    bufs = (ids_a, ids_b)
    sems = (sem_a, sem_b)
    handles = [None, None]
    handles[0] = pltpu.async_copy(
        ids_hbm.at[pl.ds(base, CHUNK_ROWS), :], ids_a, sem_a)
    for ch in range(NCHUNK):
        cur = ch % 2
        handles[cur].wait()
        if ch + 1 < NCHUNK:
            nxt = (ch + 1) % 2
            handles[nxt] = pltpu.async_copy(
                ids_hbm.at[pl.ds(base + (ch + 1) * CHUNK_ROWS, CHUNK_ROWS), :],
                bufs[nxt], sems[nxt])
        ids_buf = bufs[cur]
        for blk in range(NBLK):
            rowv = lanes + blk * LANES
            zf = jnp.zeros((LANES,), jnp.float32)

            def lacc(ll, carry, ids_buf=ids_buf, rowv=rowv):
                a0, a1, lvec = carry
                idv = plsc.load_gather(ids_buf, [rowv, lvec])
                g0 = plsc.load_gather(m0_v, [idv])
                g1 = plsc.load_gather(m1_v, [idv])
                return (a0 + g0, a1 + g1, lvec + 1)

            a0, a1, _ = plsc.parallel_loop(
                0, L, unroll=8, carry=(zf, zf, zero16))(lacc)
            a0 = a0 * inv_l + b0
            a1 = a1 * inv_l + b1
            rloc2 = (lanes + blk * LANES + ch * CHUNK_ROWS) * C
            plsc.store_scatter(out_v, [rloc2], a0)
            plsc.store_scatter(out_v, [rloc2 + 1], a1)

    pltpu.async_copy(out_v, out_hbm.at[pl.ds(base * C, ROWS_PER_W * C)],
                     sem_s).wait()


_sc_kernel = functools.partial(
    pl.kernel,
    out_type=jax.ShapeDtypeStruct((B * C,), jnp.float32),
    mesh=plsc.VectorSubcoreMesh(core_axis_name="c", subcore_axis_name="s"),
    compiler_params=pltpu.CompilerParams(needs_layout_passes=False),
    scratch_types=[
        pltpu.VMEM((CHUNK_ROWS, L), jnp.int32),
        pltpu.VMEM((CHUNK_ROWS, L), jnp.int32),
        pltpu.VMEM((V * D,), jnp.float32),
        pltpu.VMEM((C * D,), jnp.float32),
        pltpu.VMEM((LANES,), jnp.float32),
        pltpu.VMEM((VPAD,), jnp.float32),
        pltpu.VMEM((VPAD,), jnp.float32),
        pltpu.VMEM((ROWS_PER_W * C,), jnp.float32),
        pltpu.SemaphoreType.DMA,
        pltpu.SemaphoreType.DMA,
        pltpu.SemaphoreType.DMA,
    ],
)(_sc_body)


def kernel(input_ids, embed_table, W, b):
    ids = input_ids.astype(jnp.int32)
    tab = embed_table.astype(jnp.float32).reshape(V * D)
    w = W.astype(jnp.float32).reshape(C * D)
    b_pad = jnp.zeros((LANES,), jnp.float32).at[:C].set(b.astype(jnp.float32))
    out = _sc_kernel(ids, tab, w, b_pad)
    return out.reshape(B, C)


# speedup vs baseline: 141.5460x; 1.1258x over previous
import functools

import jax
import jax.numpy as jnp
from jax import lax
from jax.experimental import pallas as pl
from jax.experimental.pallas import tpu as pltpu
from jax.experimental.pallas import tpu_sc as plsc

B = 16384
L = 200
V = 200
D = 16
C = 2

_info = plsc.get_sparse_core_info()
NC, NS, LANES = _info.num_cores, _info.num_subcores, _info.num_lanes
NW = NC * NS
ROWS_PER_W = B // NW
CHUNK_ROWS = 64
NCHUNK = ROWS_PER_W // CHUNK_ROWS
NBLK = CHUNK_ROWS // LANES
VPAD = 208


def _sc_body(ids_hbm, tab_hbm, w_hbm, b_hbm, out_hbm,
             ids_a, ids_b, tab_v, w_v, b_v, m0_v, m1_v, out_v,
             sem_a, sem_b, sem_s):
    wid = lax.axis_index("s") * NC + lax.axis_index("c")
    base = wid * ROWS_PER_W

    pltpu.async_copy(tab_hbm, tab_v, sem_s).wait()
    pltpu.async_copy(w_hbm, w_v, sem_s).wait()
    pltpu.async_copy(b_hbm, b_v, sem_s).wait()

    lanes = lax.iota(jnp.int32, LANES)
    zero16 = jnp.zeros((LANES,), jnp.int32)
    one16 = zero16 + 1

    for vc in range(VPAD // LANES):
        vflat = jnp.minimum(lanes + vc * LANES, V - 1) * D

        def mbody(dd, carry, vflat=vflat):
            m0, m1 = carry
            dvec = jnp.full((LANES,), dd, dtype=jnp.int32)
            col = plsc.load_gather(tab_v, [vflat + dvec])
            w0 = plsc.load_gather(w_v, [dvec])
            w1 = plsc.load_gather(w_v, [dvec + D])
            return (m0 + col * w0, m1 + col * w1)

        m0, m1 = lax.fori_loop(
            0, D, mbody,
            (jnp.zeros((LANES,), jnp.float32), jnp.zeros((LANES,), jnp.float32)))
        m0_v[pl.ds(vc * LANES, LANES)] = m0
        m1_v[pl.ds(vc * LANES, LANES)] = m1

    b0 = plsc.load_gather(b_v, [zero16])
    b1 = plsc.load_gather(b_v, [one16])
    inv_l = jnp.full((LANES,), 1.0 / L, dtype=jnp.float32)

    bufs = (ids_a, ids_b)
    sems = (sem_a, sem_b)
    handles = [None, None]
    handles[0] = pltpu.async_copy(
        ids_hbm.at[pl.ds(base * L, CHUNK_ROWS * L)], ids_a, sem_a)
    for ch in range(NCHUNK):
        cur = ch % 2
        handles[cur].wait()
        if ch + 1 < NCHUNK:
            nxt = (ch + 1) % 2
            handles[nxt] = pltpu.async_copy(
                ids_hbm.at[pl.ds((base + (ch + 1) * CHUNK_ROWS) * L,
                                 CHUNK_ROWS * L)],
                bufs[nxt], sems[nxt])
        ids_buf = bufs[cur]
        for blk in range(NBLK):
            rowbase = (lanes + blk * LANES) * L
            zf = jnp.zeros((LANES,), jnp.float32)
            UNROLL = 8

            def lacc(it, carry, ids_buf=ids_buf, rowbase=rowbase):
                accs = list(carry)
                lvec = rowbase + jnp.full((LANES,), it * UNROLL,
                                          dtype=jnp.int32)
                for k in range(UNROLL):
                    idv = plsc.load_gather(ids_buf, [lvec + k])
                    g0 = plsc.load_gather(m0_v, [idv])
                    g1 = plsc.load_gather(m1_v, [idv])
                    accs[2 * (k % 2)] = accs[2 * (k % 2)] + g0
                    accs[2 * (k % 2) + 1] = accs[2 * (k % 2) + 1] + g1
                return tuple(accs)

            a0e, a1e, a0o, a1o = lax.fori_loop(
                0, L // UNROLL, lacc, (zf, zf, zf, zf))
            a0 = (a0e + a0o) * inv_l + b0
            a1 = (a1e + a1o) * inv_l + b1
            rloc2 = (lanes + blk * LANES + ch * CHUNK_ROWS) * C
            plsc.store_scatter(out_v, [rloc2], a0)
            plsc.store_scatter(out_v, [rloc2 + 1], a1)

    pltpu.async_copy(out_v, out_hbm.at[pl.ds(base * C, ROWS_PER_W * C)],
                     sem_s).wait()


_sc_kernel = functools.partial(
    pl.kernel,
    out_type=jax.ShapeDtypeStruct((B * C,), jnp.float32),
    mesh=plsc.VectorSubcoreMesh(core_axis_name="c", subcore_axis_name="s"),
    compiler_params=pltpu.CompilerParams(needs_layout_passes=False),
    scratch_types=[
        pltpu.VMEM((CHUNK_ROWS * L,), jnp.int32),
        pltpu.VMEM((CHUNK_ROWS * L,), jnp.int32),
        pltpu.VMEM((V * D,), jnp.float32),
        pltpu.VMEM((C * D,), jnp.float32),
        pltpu.VMEM((LANES,), jnp.float32),
        pltpu.VMEM((VPAD,), jnp.float32),
        pltpu.VMEM((VPAD,), jnp.float32),
        pltpu.VMEM((ROWS_PER_W * C,), jnp.float32),
        pltpu.SemaphoreType.DMA,
        pltpu.SemaphoreType.DMA,
        pltpu.SemaphoreType.DMA,
    ],
)(_sc_body)


def kernel(input_ids, embed_table, W, b):
    ids = input_ids.astype(jnp.int32).reshape(B * L)
    tab = embed_table.astype(jnp.float32).reshape(V * D)
    w = W.astype(jnp.float32).reshape(C * D)
    b_pad = jnp.zeros((LANES,), jnp.float32).at[:C].set(b.astype(jnp.float32))
    out = _sc_kernel(ids, tab, w, b_pad)
    return out.reshape(B, C)
